# baseline (speedup 1.0000x reference)
import jax
import jax.numpy as jnp
from jax import lax
from jax.experimental import pallas as pl
from jax.experimental.pallas import tpu as pltpu
from jax.experimental.pallas import tpu_sc as plsc

VOCAB = 1000000
EMBED_DIM = 64
BATCH = 16384
FIELDS = 26

NC = 2
NS = 16
NW = NC * NS

NBT = BATCH // 128
NTC = FIELDS * NBT
PER_W = NTC // NW
LANE = 128


def _transpose_tile(rows_v, cols_v, pregs):
    @plsc.parallel_loop(0, LANE, unroll=8)
    def _(l):
        for g in range(4):
            vec = rows_v[l, pl.ds(g * 16, 16)]
            plsc.store_scatter(cols_v, [pregs[g] + l], vec)


def _body(weight_hbm, idx_hbm, out_hbm,
          idx_v, rows0, rows1, cols0, cols1, sg0, sg1, so0, so1):
    wid = lax.axis_index("s") * NC + lax.axis_index("c")
    pltpu.sync_copy(idx_hbm.at[wid], idx_v)
    iota16 = lax.broadcasted_iota(jnp.int32, (16,), 0)
    pregs = [(iota16 + g * 16) * 128 for g in range(4)]

    rows = (rows0, rows1)
    cols = (cols0, cols1)
    sg = (sg0, sg1)
    so = (so0, so1)

    pltpu.async_copy(weight_hbm.at[idx_v.at[0]], rows0, sg0)
    pltpu.async_copy(weight_hbm.at[idx_v.at[1]], rows1, sg1)

    def pair(it, carry):
        for p in range(2):
            j = 2 * it + p
            t = wid * PER_W + j
            f = t // NBT
            bt = t - f * NBT
            pltpu.make_async_copy(
                weight_hbm.at[idx_v.at[j]], rows[p], sg[p]).wait()
            @pl.when(j >= 2)
            def _():
                for dt in range(8):
                    pltpu.make_async_copy(
                        cols[p].at[pl.ds(dt * 1024, 1024)],
                        out_hbm.at[f, dt, bt], so[p]).wait()
            _transpose_tile(rows[p], cols[p], pregs)
            @pl.when(j + 2 < PER_W)
            def _():
                pltpu.async_copy(
                    weight_hbm.at[idx_v.at[j + 2]], rows[p], sg[p])
            for dt in range(8):
                pltpu.async_copy(cols[p].at[pl.ds(dt * 1024, 1024)],
                                 out_hbm.at[f, dt, bt], so[p])
        return carry

    lax.fori_loop(0, PER_W // 2, pair, 0)

    for p in range(2):
        j = PER_W - 2 + p
        t = wid * PER_W + j
        f = t // NBT
        bt = t - f * NBT
        for dt in range(8):
            pltpu.make_async_copy(cols[p].at[pl.ds(dt * 1024, 1024)],
                                  out_hbm.at[f, dt, bt], so[p]).wait()


@jax.jit
def _embed(idx, weight):
    mesh = plsc.VectorSubcoreMesh(core_axis_name="c", subcore_axis_name="s")
    k = pl.kernel(
        _body,
        out_type=jax.ShapeDtypeStruct((FIELDS, 8, NBT, 1024), jnp.float32),
        mesh=mesh,
        scratch_types=[
            pltpu.VMEM((PER_W, LANE), jnp.int32),
            pltpu.VMEM((LANE, EMBED_DIM), jnp.float32),
            pltpu.VMEM((LANE, EMBED_DIM), jnp.float32),
            pltpu.VMEM((8192,), jnp.float32),
            pltpu.VMEM((8192,), jnp.float32),
            pltpu.SemaphoreType.DMA,
            pltpu.SemaphoreType.DMA,
            pltpu.SemaphoreType.DMA,
            pltpu.SemaphoreType.DMA,
        ],
        compiler_params=pltpu.CompilerParams(
            use_tc_tiling_on_sc=False, needs_layout_passes=False),
    )
    return k(weight, idx)


def kernel(input, weight):
    idx = input.astype(jnp.int32).T.reshape(NW, PER_W, LANE)
    out5 = _embed(idx, weight)
    return (out5.reshape(FIELDS, 8, NBT, 8, LANE)
            .transpose(2, 4, 0, 1, 3)
            .reshape(BATCH, FIELDS, EMBED_DIM))


# speedup vs baseline: 1.4023x; 1.4023x over previous
import jax
import jax.numpy as jnp
from jax import lax
from jax.experimental import pallas as pl
from jax.experimental.pallas import tpu as pltpu
from jax.experimental.pallas import tpu_sc as plsc

VOCAB = 1000000
EMBED_DIM = 64
BATCH = 16384
FIELDS = 26

NC = 2
NS = 16
NW = NC * NS

NBT = BATCH // 128
NTC = FIELDS * NBT
PER_W = NTC // NW
LANE = 128


def _transpose_tile(rows_v, cols_v, dixs):
    @plsc.parallel_loop(0, LANE, unroll=8)
    def _(l):
        ls = jnp.full((16,), l, jnp.int32)
        for g in range(4):
            vec = rows_v[l, pl.ds(g * 16, 16)]
            plsc.store_scatter(cols_v, [dixs[g], ls], vec)


def _body(weight_hbm, idx_hbm, out_hbm,
          idx_v, rows0, rows1, cols0, cols1, sg0, sg1, so0, so1):
    wid = lax.axis_index("s") * NC + lax.axis_index("c")
    pltpu.sync_copy(idx_hbm.at[wid], idx_v)
    iota16 = lax.broadcasted_iota(jnp.int32, (16,), 0)
    dixs = [iota16 + g * 16 for g in range(4)]

    rows = (rows0, rows1)
    cols = (cols0, cols1)
    sg = (sg0, sg1)
    so = (so0, so1)

    pltpu.async_copy(weight_hbm.at[idx_v.at[0]], rows0, sg0)
    pltpu.async_copy(weight_hbm.at[idx_v.at[1]], rows1, sg1)

    def pair(it, carry):
        for p in range(2):
            j = 2 * it + p
            t = wid * PER_W + j
            f = t // NBT
            bt = t - f * NBT
            pltpu.make_async_copy(
                weight_hbm.at[idx_v.at[j]], rows[p], sg[p]).wait()
            @pl.when(j >= 2)
            def _():
                for dt in range(8):
                    pltpu.make_async_copy(
                        cols[p].at[pl.ds(dt * 8, 8), pl.ds(0, LANE)],
                        out_hbm.at[f, dt, bt], so[p]).wait()
            _transpose_tile(rows[p], cols[p], dixs)
            @pl.when(j + 2 < PER_W)
            def _():
                pltpu.async_copy(
                    weight_hbm.at[idx_v.at[j + 2]], rows[p], sg[p])
            for dt in range(8):
                pltpu.async_copy(cols[p].at[pl.ds(dt * 8, 8), pl.ds(0, LANE)],
                                 out_hbm.at[f, dt, bt], so[p])
        return carry

    lax.fori_loop(0, PER_W // 2, pair, 0)

    for p in range(2):
        j = PER_W - 2 + p
        t = wid * PER_W + j
        f = t // NBT
        bt = t - f * NBT
        for dt in range(8):
            pltpu.make_async_copy(cols[p].at[pl.ds(dt * 8, 8), pl.ds(0, LANE)],
                                  out_hbm.at[f, dt, bt], so[p]).wait()


@jax.jit
def _embed(idx, weight):
    mesh = plsc.VectorSubcoreMesh(core_axis_name="c", subcore_axis_name="s")
    k = pl.kernel(
        _body,
        out_type=jax.ShapeDtypeStruct((FIELDS, 8, NBT, 8, LANE), jnp.float32),
        mesh=mesh,
        scratch_types=[
            pltpu.VMEM((PER_W, LANE), jnp.int32),
            pltpu.VMEM((LANE, EMBED_DIM), jnp.float32),
            pltpu.VMEM((LANE, EMBED_DIM), jnp.float32),
            pltpu.VMEM((EMBED_DIM, 129), jnp.float32),
            pltpu.VMEM((EMBED_DIM, 129), jnp.float32),
            pltpu.SemaphoreType.DMA,
            pltpu.SemaphoreType.DMA,
            pltpu.SemaphoreType.DMA,
            pltpu.SemaphoreType.DMA,
        ],
        compiler_params=pltpu.CompilerParams(
            use_tc_tiling_on_sc=False, needs_layout_passes=False),
    )
    return k(weight, idx)


def kernel(input, weight):
    idx = input.astype(jnp.int32).T.reshape(NW, PER_W, LANE)
    out5 = _embed(idx, weight)
    return (out5.transpose(2, 4, 0, 1, 3)
            .reshape(BATCH, FIELDS, EMBED_DIM))
